# baseline (speedup 1.0000x reference)
import functools

import jax
import jax.numpy as jnp
from jax import lax
from jax.experimental import pallas as pl
from jax.experimental.pallas import tpu as pltpu
from jax.experimental.pallas import tpu_sc as plsc

SEQ_LEN = 200
EMBED_DIM = 64
BATCH = 4096
VOCAB = 1000000

NC, NS, L = 2, 16, 16
NW = NC * NS
BBLK = BATCH // NW
TROW, TCOL = SEQ_LEN // 8, BATCH // 128
NBUF = 4
SCALE = 8.0


K0_W = 384
K0_BLOCKS = (VOCAB - 64) // K0_W
K0_REM = K0_BLOCKS - NW * (K0_BLOCKS // NW)
K0_PER_W = K0_BLOCKS // NW
K0_NITER = K0_PER_W + 3


@functools.partial(
    pl.kernel,
    out_type=jax.ShapeDtypeStruct((VOCAB // 16, 8, 128), jnp.float32),
    mesh=plsc.VectorSubcoreMesh(core_axis_name="c", subcore_axis_name="s"),
    compiler_params=pltpu.CompilerParams(
        use_tc_tiling_on_sc=True, needs_layout_passes=False),
    scratch_types=[
        [pltpu.VMEM((EMBED_DIM, K0_W), jnp.float32) for _ in range(2)],
        [pltpu.VMEM((K0_W // 16, 8, 128), jnp.float32) for _ in range(2)],
        pltpu.VMEM((4, 8, 128), jnp.float32),
        [pltpu.SemaphoreType.DMA for _ in range(2)],
        [pltpu.SemaphoreType.DMA for _ in range(2)],
    ],
)
def _depad_kernel(ttT_hbm, tail3_hbm, out_hbm, in_v, out_v, tail_v, isem, wsem):
    wid = lax.axis_index("s") * NC + lax.axis_index("c")
    base = wid * K0_PER_W + jnp.minimum(wid, K0_REM)
    n_w = K0_PER_W + (wid < K0_REM).astype(jnp.int32)

    def v0_of(i):
        return pl.multiple_of(K0_W * (base + jnp.minimum(i, n_w - 1)), 128)

    def in_descr(v0, s):
        return pltpu.make_async_copy(
            ttT_hbm.at[:, pl.ds(v0, K0_W)], in_v[s], isem[s])

    def out_descr(v0, s):
        return pltpu.make_async_copy(
            out_v[s],
            out_hbm.at[pl.ds(lax.shift_right_logical(v0, 4), K0_W // 16)],
            wsem[s])

    iota = lax.iota(jnp.int32, L)
    in_descr(v0_of(0), 0).start()

    def body(i, carry):
        for s in range(2):
            i2 = i * 2 + s
            ns = 1 - s

            @pl.when(i2 + 1 < K0_NITER)
            def _():
                in_descr(v0_of(i2 + 1), ns).start()

            in_descr(v0_of(i2), s).wait()

            @pl.when(i2 >= 2)
            def _():
                out_descr(v0_of(i2 - 2), s).wait()

            @plsc.parallel_loop(0, K0_W * 4, unroll=4)
            def _(o):
                r = lax.shift_right_logical(o, 6)
                sr = jnp.bitwise_and(lax.shift_right_logical(o, 3), 7)
                l0 = jnp.bitwise_and(o, 7) * L
                rowi = jnp.bitwise_and(o, 3) * L + iota
                coli = jnp.full((L,), r * 16 + sr * 2
                                + jnp.bitwise_and(lax.shift_right_logical(o, 2), 1),
                                jnp.int32)
                out_v[s][r, sr, pl.ds(l0, L)] = plsc.load_gather(in_v[s], [rowi, coli])

            out_descr(v0_of(i2), s).start()
        return carry

    lax.fori_loop(0, K0_NITER // 2, body, 0)
    out_descr(v0_of(K0_NITER - 2), 0).wait()
    out_descr(v0_of(K0_NITER - 1), 1).wait()

    @pl.when(wid == NW - 1)
    def _():
        pltpu.sync_copy(tail3_hbm, tail_v)
        pltpu.sync_copy(tail_v, out_hbm.at[pl.ds(VOCAB // 16 - 4, 4)])


@functools.partial(
    pl.kernel,
    out_type=jax.ShapeDtypeStruct((SEQ_LEN, 8, TCOL, 8, 128), jnp.float32),
    mesh=plsc.VectorSubcoreMesh(core_axis_name="c", subcore_axis_name="s"),
    compiler_params=pltpu.CompilerParams(
        use_tc_tiling_on_sc=False, needs_layout_passes=False),
    scratch_types=[
        [pltpu.VMEM((BBLK,), jnp.int32) for _ in range(NBUF)],
        [pltpu.VMEM((BBLK, EMBED_DIM), jnp.float32) for _ in range(NBUF)],
        [pltpu.VMEM((8, 8, 128), jnp.float32) for _ in range(NBUF)],
        pltpu.VMEM((SEQ_LEN, EMBED_DIM), jnp.float32),
        [pltpu.SemaphoreType.DMA for _ in range(NBUF)],
        [pltpu.SemaphoreType.DMA for _ in range(NBUF)],
        [pltpu.SemaphoreType.DMA for _ in range(NBUF)],
    ],
)
def _embed_kernel(idx4_hbm, table_hbm, pos_hbm, out_hbm,
                  idx_v, rows_v, outb_v, pos_v,
                  isem, gsem, wsem):
    wid = lax.axis_index("s") * NC + lax.axis_index("c")

    pltpu.sync_copy(pos_hbm, pos_v)

    def idx_descr(t, b):
        return pltpu.make_async_copy(
            idx4_hbm.at[t // 8, wid, t % 8], idx_v[b], isem[b])

    def gather_descr(b):
        return pltpu.make_async_copy(
            table_hbm.at[idx_v[b]], rows_v[b], gsem[b])

    def wb_descrs(t, b):
        return [pltpu.make_async_copy(
            outb_v[b].at[er], out_hbm.at[t, er, wid], wsem[b])
            for er in range(8)]

    idx_descr(0, 0).start()
    idx_descr(1, 1).start()
    idx_descr(2, 2).start()
    idx_descr(0, 0).wait()
    gather_descr(0).start()
    idx_descr(1, 1).wait()
    gather_descr(1).start()

    iota = lax.iota(jnp.int32, L)

    def outer(o, carry):
        for b in range(NBUF):
            t = o * NBUF + b

            @pl.when(t + 3 < SEQ_LEN)
            def _():
                idx_descr(t + 3, (b + 3) % NBUF).start()

            @pl.when(t + 2 < SEQ_LEN)
            def _():
                nb = (b + 2) % NBUF
                idx_descr(t + 2, nb).wait()
                gather_descr(nb).start()

            gather_descr(b).wait()

            @pl.when(t >= NBUF)
            def _():
                for d in wb_descrs(t - NBUF, b):
                    d.wait()

            t_vec = jnp.full((L,), t, jnp.int32)

            @plsc.parallel_loop(0, EMBED_DIM, unroll=4)
            def _(e):
                e_vec = jnp.full((L,), e, jnp.int32)
                ps = plsc.load_gather(pos_v, [t_vec, e_vec])
                er = lax.shift_right_logical(e, 3)
                es = jnp.bitwise_and(e, 7)
                for j in range(BBLK // L):
                    rowi = iota + (j * L)
                    vals = plsc.load_gather(rows_v[b], [rowi, e_vec])
                    outb_v[b][er, es, pl.ds(j * L, L)] = vals * SCALE + ps

            for d in wb_descrs(t, b):
                d.start()
        return carry

    lax.fori_loop(0, SEQ_LEN // NBUF, outer, 0)

    for b in range(NBUF):
        for d in wb_descrs(SEQ_LEN - NBUF + b, b):
            d.wait()


def kernel(inputs, token_table, pos_table):
    idx4 = inputs.T.reshape(TROW, 8, TCOL, 128).transpose(0, 2, 1, 3)
    tail3 = token_table[VOCAB - 64:].reshape(4, 8, 128)
    table_lin = _depad_kernel(token_table.T, tail3).reshape(VOCAB, EMBED_DIM)
    out5 = _embed_kernel(idx4, table_lin, pos_table)
    return out5.transpose(2, 4, 0, 1, 3).reshape(BATCH, SEQ_LEN, EMBED_DIM)


# speedup vs baseline: 1.3928x; 1.3928x over previous
import functools

import jax
import jax.numpy as jnp
from jax import lax
from jax.experimental import pallas as pl
from jax.experimental.pallas import tpu as pltpu
from jax.experimental.pallas import tpu_sc as plsc

SEQ_LEN = 200
EMBED_DIM = 64
BATCH = 4096
VOCAB = 1000000

NC, NS, L = 2, 16, 16
NW = NC * NS
BBLK = BATCH // NW
TROW, TCOL = SEQ_LEN // 8, BATCH // 128
NBUF = 4
SCALE = 8.0


@functools.partial(
    pl.kernel,
    out_type=jax.ShapeDtypeStruct((SEQ_LEN, 8, TCOL, 8, 128), jnp.float32),
    mesh=plsc.VectorSubcoreMesh(core_axis_name="c", subcore_axis_name="s"),
    compiler_params=pltpu.CompilerParams(
        use_tc_tiling_on_sc=False, needs_layout_passes=False),
    scratch_types=[
        [pltpu.VMEM((BBLK,), jnp.int32) for _ in range(NBUF)],
        [pltpu.VMEM((BBLK, EMBED_DIM), jnp.float32) for _ in range(NBUF)],
        [pltpu.VMEM((8, 8, 128), jnp.float32) for _ in range(NBUF)],
        pltpu.VMEM((SEQ_LEN, EMBED_DIM), jnp.float32),
        [pltpu.SemaphoreType.DMA for _ in range(NBUF)],
        [pltpu.SemaphoreType.DMA for _ in range(NBUF)],
        [pltpu.SemaphoreType.DMA for _ in range(NBUF)],
    ],
)
def _embed_kernel(idx4_hbm, table_hbm, pos_hbm, out_hbm,
                  idx_v, rows_v, outb_v, pos_v,
                  isem, gsem, wsem):
    wid = lax.axis_index("s") * NC + lax.axis_index("c")

    pltpu.sync_copy(pos_hbm, pos_v)

    def idx_descr(t, b):
        return pltpu.make_async_copy(
            idx4_hbm.at[t // 8, wid, t % 8], idx_v[b], isem[b])

    def gather_descr(b):
        return pltpu.make_async_copy(
            table_hbm.at[idx_v[b]], rows_v[b], gsem[b])

    def wb_descrs(t, b):
        return [pltpu.make_async_copy(
            outb_v[b].at[er], out_hbm.at[t, er, wid], wsem[b])
            for er in range(8)]

    idx_descr(0, 0).start()
    idx_descr(1, 1).start()
    idx_descr(2, 2).start()
    idx_descr(0, 0).wait()
    gather_descr(0).start()
    idx_descr(1, 1).wait()
    gather_descr(1).start()

    iota = lax.iota(jnp.int32, L)

    def outer(o, carry):
        for b in range(NBUF):
            t = o * NBUF + b

            @pl.when(t + 3 < SEQ_LEN)
            def _():
                idx_descr(t + 3, (b + 3) % NBUF).start()

            @pl.when(t + 2 < SEQ_LEN)
            def _():
                nb = (b + 2) % NBUF
                idx_descr(t + 2, nb).wait()
                gather_descr(nb).start()

            gather_descr(b).wait()

            @pl.when(t >= NBUF)
            def _():
                for d in wb_descrs(t - NBUF, b):
                    d.wait()

            t_vec = jnp.full((L,), t, jnp.int32)

            @plsc.parallel_loop(0, EMBED_DIM, unroll=4)
            def _(e):
                e_vec = jnp.full((L,), e, jnp.int32)
                ps = plsc.load_gather(pos_v, [t_vec, e_vec])
                er = lax.shift_right_logical(e, 3)
                es = jnp.bitwise_and(e, 7)
                for j in range(BBLK // L):
                    rowi = iota + (j * L)
                    vals = plsc.load_gather(rows_v[b], [rowi, e_vec])
                    outb_v[b][er, es, pl.ds(j * L, L)] = vals * SCALE + ps

            for d in wb_descrs(t, b):
                d.start()
        return carry

    lax.fori_loop(0, SEQ_LEN // NBUF, outer, 0)

    for b in range(NBUF):
        for d in wb_descrs(SEQ_LEN - NBUF + b, b):
            d.wait()


def kernel(inputs, token_table, pos_table):
    idx4 = inputs.T.reshape(TROW, 8, TCOL, 128).transpose(0, 2, 1, 3)
    out5 = _embed_kernel(idx4, token_table, pos_table)
    return out5.transpose(2, 4, 0, 1, 3).reshape(BATCH, SEQ_LEN, EMBED_DIM)
